# Initial kernel scaffold; baseline (speedup 1.0000x reference)
import jax
import jax.numpy as jnp
from jax.experimental import pallas as pl


def kernel(h, edge_index, W, a_l, a_r, bias):
    raise NotImplementedError("write your pallas kernel here")


# speedup vs baseline: 5.8648x; 5.8648x over previous
import functools

import jax
import jax.numpy as jnp
from jax import lax
from jax.experimental import pallas as pl
from jax.experimental.pallas import tpu as pltpu
from jax.experimental.pallas import tpu_sc as plsc

N = 10000
E = 160000
IN_DIM = 256
H = 8
D = 32
NEG_SLOPE = 0.2

NPAD = 10112
ROWS_PER_TILE = NPAD // 16
EDGES_PER_TILE = E // 16
CHUNK = 80
NCHUNKS = EDGES_PER_TILE // CHUNK


def _proj_body(h_ref, w_ref, al_ref, ar_ref, alc_ref, arc_ref,
               feat_ref, els_ref, erd_ref, elc_ref, erc_ref):
    feat = jnp.dot(h_ref[...], w_ref[...], preferred_element_type=jnp.float32)
    elx = jnp.dot(feat, al_ref[...], preferred_element_type=jnp.float32)
    erx = jnp.dot(feat, ar_ref[...], preferred_element_type=jnp.float32)
    elc = jnp.dot(feat, alc_ref[...], preferred_element_type=jnp.float32)
    erc = jnp.dot(feat, arc_ref[...], preferred_element_type=jnp.float32)
    feat_ref[0] = feat[:, :128]
    feat_ref[1] = feat[:, 128:]
    els_ref[0] = elx[:, :128]
    els_ref[1] = elx[:, 128:]
    erd_ref[0] = erx[:, :128]
    erd_ref[1] = erx[:, 128:]
    elc_ref[0] = elc[:, :128]
    elc_ref[1] = elc[:, 128:]
    erc_ref[0] = erc[:, :128]
    erc_ref[1] = erc[:, 128:]


def _project(h, W, alx, arx, alc, arc):
    blk = 1000
    grid = (N // blk,)
    big = jax.ShapeDtypeStruct((2, N, 128), jnp.float32)
    sml = jax.ShapeDtypeStruct((2, N, 128), jnp.float32)
    return pl.pallas_call(
        _proj_body,
        grid=grid,
        in_specs=[
            pl.BlockSpec((blk, IN_DIM), lambda i: (i, 0)),
            pl.BlockSpec((IN_DIM, H * D), lambda i: (0, 0)),
            pl.BlockSpec((H * D, H * D), lambda i: (0, 0)),
            pl.BlockSpec((H * D, H * D), lambda i: (0, 0)),
            pl.BlockSpec((H * D, H * D), lambda i: (0, 0)),
            pl.BlockSpec((H * D, H * D), lambda i: (0, 0)),
        ],
        out_specs=[
            pl.BlockSpec((2, blk, 128), lambda i: (0, i, 0)),
            pl.BlockSpec((2, blk, 128), lambda i: (0, i, 0)),
            pl.BlockSpec((2, blk, 128), lambda i: (0, i, 0)),
            pl.BlockSpec((2, blk, 128), lambda i: (0, i, 0)),
            pl.BlockSpec((2, blk, 128), lambda i: (0, i, 0)),
        ],
        out_shape=[big, big, big, sml, sml],
    )(h, W, alx, arx, alc, arc)


def _num_body(featx, elsx, erdx, src_hbm, dst_hbm, acc_out,
              src_v, dst_v, gidx_v, rows_v, els_v, erd_v, acc_sh, sem):
    cid = lax.axis_index("c")
    sid = lax.axis_index("s")
    rbase = sid * ROWS_PER_TILE
    z16 = jnp.zeros((16,), jnp.float32)
    for e in range(CHUNK):
        for j in range(8):
            rows_v[e, pl.ds(16 * j, 16)] = z16
    for k in range(7):
        pltpu.sync_copy(rows_v, acc_sh.at[pl.ds(rbase + k * 80, 80)])
    pltpu.sync_copy(rows_v.at[pl.ds(0, 72)],
                    acc_sh.at[pl.ds(rbase + 560, 72)])
    plsc.subcore_barrier()

    ebase = sid * EDGES_PER_TILE
    fbase = cid * N

    def chunk_body(c, carry):
        off = ebase + c * CHUNK
        pltpu.sync_copy(src_hbm.at[pl.ds(off, CHUNK)], src_v)
        pltpu.sync_copy(dst_hbm.at[pl.ds(off, CHUNK)], dst_v)
        for g in range(CHUNK // 16):
            gidx_v[pl.ds(g * 16, 16)] = src_v[pl.ds(g * 16, 16)] + fbase
        pltpu.async_copy(featx.at[gidx_v], rows_v, sem).wait()
        pltpu.async_copy(elsx.at[gidx_v], els_v, sem).wait()
        for g in range(CHUNK // 16):
            gidx_v[pl.ds(g * 16, 16)] = dst_v[pl.ds(g * 16, 16)] + fbase
        pltpu.async_copy(erdx.at[gidx_v], erd_v, sem).wait()

        def edge_body(e, cc):
            for j in range(8):
                sl = pl.ds(16 * j, 16)
                x = els_v[e, sl] + erd_v[e, sl]
                x = jnp.where(x >= 0.0, x, x * NEG_SLOPE)
                rows_v[e, sl] = rows_v[e, sl] * jnp.exp(x)
            return cc

        lax.fori_loop(0, CHUNK, edge_body, 0)
        pltpu.sync_copy(rows_v, acc_sh.at[dst_v], add=True)
        return carry

    lax.fori_loop(0, NCHUNKS, chunk_body, 0)
    plsc.subcore_barrier()
    wid = cid * 16 + sid
    pltpu.sync_copy(acc_sh.at[pl.ds(rbase, ROWS_PER_TILE)], acc_out.at[wid])


def _num_pass(featx, elsx, erdx, src, dst):
    mesh = plsc.VectorSubcoreMesh(core_axis_name="c", subcore_axis_name="s")
    fn = functools.partial(
        pl.kernel,
        mesh=mesh,
        out_type=jax.ShapeDtypeStruct((32, ROWS_PER_TILE, 128), jnp.float32),
        scratch_types=[
            pltpu.VMEM((CHUNK,), jnp.int32),
            pltpu.VMEM((CHUNK,), jnp.int32),
            pltpu.VMEM((CHUNK,), jnp.int32),
            pltpu.VMEM((CHUNK, 128), jnp.float32),
            pltpu.VMEM((CHUNK, 128), jnp.float32),
            pltpu.VMEM((CHUNK, 128), jnp.float32),
            pltpu.VMEM_SHARED((NPAD, 128), jnp.float32),
            pltpu.SemaphoreType.DMA,
        ],
    )(_num_body)
    return fn(featx, elsx, erdx, src, dst)


def _den_body(elcx, ercx, src_hbm, dst_hbm, den_out,
              src_v, dst_v, gidx_v, elc_v, erc_v, wd_v, den_sh, sem):
    cid = lax.axis_index("c")
    sid = lax.axis_index("s")
    rbase = sid * ROWS_PER_TILE
    z16 = jnp.zeros((16,), jnp.float32)
    for e in range(CHUNK):
        for j in range(8):
            wd_v[e, pl.ds(16 * j, 16)] = z16
    for k in range(7):
        pltpu.sync_copy(wd_v, den_sh.at[pl.ds(rbase + k * 80, 80)])
    pltpu.sync_copy(wd_v.at[pl.ds(0, 72)],
                    den_sh.at[pl.ds(rbase + 560, 72)])
    plsc.subcore_barrier()

    lane = lax.iota(jnp.int32, 16)
    hmask = lane < 4
    ebase = sid * EDGES_PER_TILE
    fbase = cid * N

    def chunk_body(c, carry):
        off = ebase + c * CHUNK
        pltpu.sync_copy(src_hbm.at[pl.ds(off, CHUNK)], src_v)
        pltpu.sync_copy(dst_hbm.at[pl.ds(off, CHUNK)], dst_v)
        for g in range(CHUNK // 16):
            gidx_v[pl.ds(g * 16, 16)] = src_v[pl.ds(g * 16, 16)] + fbase
        pltpu.async_copy(elcx.at[gidx_v], elc_v, sem).wait()
        for g in range(CHUNK // 16):
            gidx_v[pl.ds(g * 16, 16)] = dst_v[pl.ds(g * 16, 16)] + fbase
        pltpu.async_copy(ercx.at[gidx_v], erc_v, sem).wait()

        def edge_body(e, cc):
            x = elc_v[e, pl.ds(0, 16)] + erc_v[e, pl.ds(0, 16)]
            x = jnp.where(x >= 0.0, x, x * NEG_SLOPE)
            wd_v[e, pl.ds(0, 16)] = jnp.where(hmask, jnp.exp(x), 0.0)
            return cc

        lax.fori_loop(0, CHUNK, edge_body, 0)
        pltpu.sync_copy(wd_v, den_sh.at[dst_v], add=True)
        return carry

    lax.fori_loop(0, NCHUNKS, chunk_body, 0)
    plsc.subcore_barrier()
    wid = cid * 16 + sid
    pltpu.sync_copy(den_sh.at[pl.ds(rbase, ROWS_PER_TILE)], den_out.at[wid])


def _den_pass(elcx, ercx, src, dst):
    mesh = plsc.VectorSubcoreMesh(core_axis_name="c", subcore_axis_name="s")
    fn = functools.partial(
        pl.kernel,
        mesh=mesh,
        out_type=jax.ShapeDtypeStruct((32, ROWS_PER_TILE, 128), jnp.float32),
        scratch_types=[
            pltpu.VMEM((CHUNK,), jnp.int32),
            pltpu.VMEM((CHUNK,), jnp.int32),
            pltpu.VMEM((CHUNK,), jnp.int32),
            pltpu.VMEM((CHUNK, 128), jnp.float32),
            pltpu.VMEM((CHUNK, 128), jnp.float32),
            pltpu.VMEM((CHUNK, 128), jnp.float32),
            pltpu.VMEM_SHARED((NPAD, 128), jnp.float32),
            pltpu.SemaphoreType.DMA,
        ],
    )(_den_body)
    return fn(elcx, ercx, src, dst)


@jax.jit
def kernel(h, edge_index, W, a_l, a_r, bias):
    gg = lax.broadcasted_iota(jnp.int32, (H * D, H * D), 0)
    cc = lax.broadcasted_iota(jnp.int32, (H * D, H * D), 1)
    alx = jnp.where(gg // D == cc // D, a_l[cc // D, gg % D], 0.0)
    arx = jnp.where(gg // D == cc // D, a_r[cc // D, gg % D], 0.0)
    chead = 4 * (cc // 128) + cc % 128
    valid = (cc % 128 < 4) & (gg // D == chead)
    alc = jnp.where(valid, a_l[jnp.clip(chead, 0, H - 1), gg % D], 0.0)
    arc = jnp.where(valid, a_r[jnp.clip(chead, 0, H - 1), gg % D], 0.0)

    featx3, elsx3, erdx3, elcx3, ercx3 = _project(h, W, alx, arx, alc, arc)
    featx = featx3.reshape(2 * N, 128)
    elsx = elsx3.reshape(2 * N, 128)
    erdx = erdx3.reshape(2 * N, 128)
    elcx = elcx3.reshape(2 * N, 128)
    ercx = ercx3.reshape(2 * N, 128)

    src = edge_index[0]
    dst = edge_index[1]
    acc = _num_pass(featx, elsx, erdx, src, dst)
    dpk = _den_pass(elcx, ercx, src, dst)

    acc = acc.reshape(2, NPAD, 128)[:, :N, :].reshape(2, N, 4, D)
    den = dpk.reshape(2, NPAD, 128)[:, :N, :4]
    out = acc / (den[..., None] + 1e-9)
    out = out.transpose(1, 0, 2, 3).reshape(N, H, D) + bias.reshape(1, H, D)
    return out
